# Initial kernel scaffold; baseline (speedup 1.0000x reference)
import jax
import jax.numpy as jnp
from jax.experimental import pallas as pl


def kernel(x, w):
    raise NotImplementedError("write your pallas kernel here")


# speedup vs baseline: 1.0440x; 1.0440x over previous
import functools
import math

import jax
import jax.numpy as jnp
from jax import lax
from jax.experimental import pallas as pl
from jax.experimental.pallas import tpu as pltpu
from jax.experimental.pallas import tpu_sc as plsc

_DIM = 32
_NW = 32
_CHUNK = 128


def _make_sc_gather(n_chunks, scale):
    mesh = plsc.VectorSubcoreMesh(core_axis_name="c", subcore_axis_name="s")

    @functools.partial(
        pl.kernel,
        mesh=mesh,
        out_type=jax.ShapeDtypeStruct((_NW, n_chunks, _CHUNK, _DIM), jnp.float32),
        scratch_types=[
            pltpu.VMEM((n_chunks, _CHUNK), jnp.int32),
            pltpu.VMEM((_CHUNK, _DIM), jnp.float32),
            pltpu.SemaphoreType.DMA,
        ],
        compiler_params=pltpu.CompilerParams(use_tc_tiling_on_sc=False),
    )
    def k(x_hbm, w_hbm, out_hbm, idx_v, rows_v, sem):
        wid = lax.axis_index("s") * 2 + lax.axis_index("c")
        pltpu.sync_copy(x_hbm.at[wid], idx_v)

        def chunk_body(j, carry):
            pltpu.async_copy(w_hbm.at[idx_v.at[j]], rows_v, sem).wait()

            def scale_body(i, c2):
                for h in range(_DIM // 16):
                    sl = pl.ds(h * 16, 16)
                    rows_v[i, sl] = rows_v[i, sl] * scale
                return c2

            lax.fori_loop(0, _CHUNK, scale_body, 0)
            pltpu.sync_copy(rows_v, out_hbm.at[wid, j])
            return carry

        lax.fori_loop(0, n_chunks, chunk_body, 0)

    return k


def kernel(x, w):
    b, h = x.shape
    n = b * h
    per_w = n // _NW
    n_chunks = per_w // _CHUNK
    scale = math.sqrt(w.shape[1])
    x3 = x.reshape(_NW, n_chunks, _CHUNK).astype(jnp.int32)
    out = _make_sc_gather(n_chunks, scale)(x3, w)
    return out.reshape(b, h, _DIM)


# speedup vs baseline: 1.2372x; 1.1850x over previous
import functools
import math

import jax
import jax.numpy as jnp
from jax import lax
from jax.experimental import pallas as pl
from jax.experimental.pallas import tpu as pltpu
from jax.experimental.pallas import tpu_sc as plsc

_DIM = 32
_NW = 32
_CHUNK = 128
_NBUF = 8
_UNROLL = 4


def _make_sc_gather(n_chunks, scale):
    mesh = plsc.VectorSubcoreMesh(core_axis_name="c", subcore_axis_name="s")
    n_groups = n_chunks // _NBUF

    @functools.partial(
        pl.kernel,
        mesh=mesh,
        out_type=jax.ShapeDtypeStruct((_NW, n_chunks, _CHUNK, _DIM), jnp.float32),
        scratch_types=[
            pltpu.VMEM((n_chunks, _CHUNK), jnp.int32),
            pltpu.VMEM((_NBUF, _CHUNK, _DIM), jnp.float32),
            pltpu.VMEM((_NBUF, _CHUNK, _DIM), jnp.float32),
            pltpu.SemaphoreType.DMA((_NBUF,)),
            pltpu.SemaphoreType.DMA((_NBUF,)),
        ],
        compiler_params=pltpu.CompilerParams(use_tc_tiling_on_sc=False),
    )
    def k(x_hbm, w_hbm, out_hbm, idx_v, gbuf, sbuf, gsem, ssem):
        wid = lax.axis_index("s") * 2 + lax.axis_index("c")
        pltpu.sync_copy(x_hbm.at[wid], idx_v)

        def gather(j, b):
            return pltpu.make_async_copy(
                w_hbm.at[idx_v.at[j]], gbuf.at[b], gsem.at[b])

        def store(j, b):
            return pltpu.make_async_copy(
                sbuf.at[b], out_hbm.at[wid, j], ssem.at[b])

        def scale_chunk(b):
            def sbody(i, c):
                for u in range(_UNROLL):
                    r = i * _UNROLL + u
                    for h in range(_DIM // 16):
                        sl = pl.ds(h * 16, 16)
                        sbuf[b, r, sl] = gbuf[b, r, sl] * scale
                return c
            lax.fori_loop(0, _CHUNK // _UNROLL, sbody, 0)

        for b in range(_NBUF):
            gather(b, b).start()
        for b in range(_NBUF):
            gather(b, b).wait()
            scale_chunk(b)
            store(b, b).start()
            gather(b + _NBUF, b).start()

        def main_body(g, c):
            for b in range(_NBUF):
                j = g * _NBUF + b
                gather(j, b).wait()
                store(j - _NBUF, b).wait()
                scale_chunk(b)
                store(j, b).start()
                gather(j + _NBUF, b).start()
            return c

        lax.fori_loop(1, n_groups - 1, main_body, 0)

        for b in range(_NBUF):
            j = (n_groups - 1) * _NBUF + b
            gather(j, b).wait()
            store(j - _NBUF, b).wait()
            scale_chunk(b)
            store(j, b).start()
        for b in range(_NBUF):
            store((n_groups - 1) * _NBUF + b, b).wait()

    return k


def kernel(x, w):
    b, h = x.shape
    n = b * h
    per_w = n // _NW
    n_chunks = per_w // _CHUNK
    scale = math.sqrt(w.shape[1])
    x3 = x.reshape(_NW, n_chunks, _CHUNK).astype(jnp.int32)
    out = _make_sc_gather(n_chunks, scale)(x3, w)
    return out.reshape(b, h, _DIM)


# speedup vs baseline: 1.3521x; 1.0928x over previous
import functools
import math

import jax
import jax.numpy as jnp
from jax import lax
from jax.experimental import pallas as pl
from jax.experimental.pallas import tpu as pltpu
from jax.experimental.pallas import tpu_sc as plsc

_DIM = 32
_NW = 32
_C = 512
_NIDX = 128
_BLK = 2048


def _make_sc_gather(n_hist, batch, scale):
    mesh = plsc.VectorSubcoreMesh(core_axis_name="c", subcore_axis_name="s")
    n_iblk = batch // _C
    assert n_iblk == _NW
    qs = _C // _NIDX

    @functools.partial(
        pl.kernel,
        mesh=mesh,
        out_type=jax.ShapeDtypeStruct((n_hist, _DIM, batch), jnp.float32),
        scratch_types=[
            pltpu.VMEM((n_hist, qs, _NIDX), jnp.int32),
            pltpu.VMEM((2, _C, _DIM), jnp.float32),
            pltpu.VMEM((2, _DIM, _C), jnp.float32),
            pltpu.SemaphoreType.DMA((2,)),
            pltpu.SemaphoreType.DMA((2,)),
            pltpu.SemaphoreType.DMA,
        ],
        compiler_params=pltpu.CompilerParams(
            use_tc_tiling_on_sc=False, needs_layout_passes=False),
    )
    def k(x_hbm, w_hbm, out_hbm, idx_v, gbuf, tbuf, gsem, ssem, isem):
        wid = lax.axis_index("s") * 2 + lax.axis_index("c")
        i0 = wid * _C
        pltpu.async_copy(
            x_hbm.at[:, pl.ds(wid * qs, qs), :], idx_v, isem).wait()

        def gath(j, b, q):
            return pltpu.make_async_copy(
                w_hbm.at[idx_v.at[j, q]],
                gbuf.at[b, pl.ds(q * _NIDX, _NIDX)],
                gsem.at[b])

        def stor(j, b):
            return pltpu.make_async_copy(
                tbuf.at[b], out_hbm.at[j, :, pl.ds(i0, _C)], ssem.at[b])

        def transpose_task(b):
            rows16 = lax.iota(jnp.int32, 16)

            def tbody(t, c):
                d = t >> 5
                c2 = t & 31
                rows = c2 * 16 + rows16
                cols = jnp.full((16,), 0, jnp.int32) + d
                tbuf[b, d, pl.ds(c2 * 16, 16)] = plsc.load_gather(
                    gbuf.at[b], [rows, cols]) * scale
                return c

            lax.fori_loop(0, _DIM * (_C // 16), tbody, 0)

        for b in range(2):
            for q in range(qs):
                gath(b, b, q).start()
        for b in range(2):
            for q in range(qs):
                gath(b, b, q).wait()
            transpose_task(b)
            stor(b, b).start()
            for q in range(qs):
                gath(b + 2, b, q).start()

        def main_body(g, c):
            for b in range(2):
                j = g * 2 + b
                for q in range(qs):
                    gath(j, b, q).wait()
                stor(j - 2, b).wait()
                transpose_task(b)
                stor(j, b).start()
                for q in range(qs):
                    gath(j + 2, b, q).start()
            return c

        lax.fori_loop(1, n_hist // 2 - 1, main_body, 0)

        for b in range(2):
            j = n_hist - 2 + b
            for q in range(qs):
                gath(j, b, q).wait()
            stor(j - 2, b).wait()
            transpose_task(b)
            stor(j, b).start()
        for b in range(2):
            stor(n_hist - 2 + b, b).wait()

    return k


def kernel(x, w):
    batch, n_hist = x.shape
    scale = math.sqrt(w.shape[1])
    x3 = x.T.reshape(n_hist, batch // _NIDX, _NIDX).astype(jnp.int32)
    out3 = _make_sc_gather(n_hist, batch, scale)(x3, w)
    return jnp.transpose(out3, (2, 0, 1))
